# baseline (speedup 1.0000x reference)
import jax
import jax.numpy as jnp
from jax.experimental import pallas as pl
from jax.experimental.pallas import tpu as pltpu

NFEAT = 128
NHID = 32
ALPHA = 0.2
N_NODES = 10000
N_CLUSTERS = 10

BR = 400
NI = N_NODES // BR


def _gcdec_kernel(x_ref, adj_ref, w_ref, b_ref, mu_ref, out_ref, q_ref,
                  support_ref):
    i = pl.program_id(0)

    @pl.when(i == 0)
    def _():
        support_ref[...] = jnp.dot(x_ref[...], w_ref[...],
                                   preferred_element_type=jnp.float32)

    o = jnp.dot(adj_ref[...], support_ref[...],
                preferred_element_type=jnp.float32) + b_ref[...]
    out_ref[...] = o

    cols = []
    for c in range(N_CLUSTERS):
        d = o - mu_ref[c:c + 1, :]
        cols.append(jnp.sum(d * d, axis=1, keepdims=True))
    dist2 = jnp.concatenate(cols, axis=1)
    qv = 1.0 / (1.0 + dist2 / ALPHA + 1e-8)
    p = jnp.exp((ALPHA + 1.0) * jnp.log(qv))
    q_ref[...] = p / jnp.sum(p, axis=1, keepdims=True)


@jax.jit
def kernel(x, adj, W, b, mu):
    b2 = b.reshape(1, NHID)
    x_hbm = pltpu.with_memory_space_constraint(x, pltpu.MemorySpace.HBM)
    out, q = pl.pallas_call(
        _gcdec_kernel,
        grid=(NI,),
        in_specs=[
            pl.BlockSpec((N_NODES, NFEAT), lambda i: (0, 0)),
            pl.BlockSpec((BR, N_NODES), lambda i: (i, 0)),
            pl.BlockSpec((NFEAT, NHID), lambda i: (0, 0)),
            pl.BlockSpec((1, NHID), lambda i: (0, 0)),
            pl.BlockSpec((N_CLUSTERS, NHID), lambda i: (0, 0)),
        ],
        out_specs=[
            pl.BlockSpec((BR, NHID), lambda i: (i, 0)),
            pl.BlockSpec((BR, N_CLUSTERS), lambda i: (i, 0)),
        ],
        out_shape=[
            jax.ShapeDtypeStruct((N_NODES, NHID), jnp.float32),
            jax.ShapeDtypeStruct((N_NODES, N_CLUSTERS), jnp.float32),
        ],
        scratch_shapes=[
            pltpu.VMEM((N_NODES, NHID), jnp.float32),
        ],
        compiler_params=pltpu.CompilerParams(
            vmem_limit_bytes=42 * 1024 * 1024,
        ),
    )(x_hbm, adj, W, b2, mu)
    return (out, q)
